# Initial kernel scaffold; baseline (speedup 1.0000x reference)
import jax
import jax.numpy as jnp
from jax.experimental import pallas as pl


def kernel(x, fc1_W, fc1_b, Wh_W, Wh_b, Wt_W, Wt_b, lin1_W, lin1_b, lin2_W, lin2_b, ln_g, ln_b):
    raise NotImplementedError("write your pallas kernel here")


# speedup vs baseline: 11.6700x; 11.6700x over previous
import functools

import jax
import jax.numpy as jnp
from jax import lax
from jax.experimental import pallas as pl
from jax.experimental.pallas import tpu as pltpu
from jax.experimental.pallas import tpu_sc as plsc

M = 4096
IN_DIM = 384
D = 512
K = 6
KPAD = 8
BLK = 256
NBLK = M // BLK
NEG = -1.0e30
SCALE = D ** -0.5


def _leaky(v):
    return jnp.where(v >= 0, v, v * 0.01)


def _mm(a, b):
    return lax.dot_general(a, b, (((1,), (0,)), ((), ())),
                           preferred_element_type=jnp.float32)


def _fc1_body(x_ref, w_ref, b_ref, h_ref, s_ref):
    i = pl.program_id(0)
    h = _leaky(_mm(x_ref[...], w_ref[...]) + b_ref[...])
    h_ref[...] = h

    @pl.when(i == 0)
    def _():
        s_ref[...] = jnp.zeros_like(s_ref)

    s_ref[...] += jnp.sum(h, axis=0, keepdims=True)


def _run_fc1(x2, fc1_W, fc1_b2):
    return pl.pallas_call(
        _fc1_body,
        grid=(NBLK,),
        in_specs=[
            pl.BlockSpec((BLK, IN_DIM), lambda i: (i, 0)),
            pl.BlockSpec((IN_DIM, D), lambda i: (0, 0)),
            pl.BlockSpec((1, D), lambda i: (0, 0)),
        ],
        out_specs=[
            pl.BlockSpec((BLK, D), lambda i: (i, 0)),
            pl.BlockSpec((1, D), lambda i: (0, 0)),
        ],
        out_shape=[
            jax.ShapeDtypeStruct((M, D), jnp.float32),
            jax.ShapeDtypeStruct((1, D), jnp.float32),
        ],
    )(x2, fc1_W, fc1_b2)


def _ehet_body(h_ref, s_ref, wh_ref, bh_ref, wt_ref, bt_ref, eh_ref, et_ref):
    h0 = (h_ref[...] + s_ref[...] * (1.0 / M)) * 0.5
    eh_ref[...] = _mm(h0, wh_ref[...]) + bh_ref[...]
    et_ref[...] = _mm(h0, wt_ref[...]) + bt_ref[...]


def _run_ehet(h0_pre, colsum, Wh_W, Wh_b2, Wt_W, Wt_b2):
    return pl.pallas_call(
        _ehet_body,
        grid=(NBLK,),
        in_specs=[
            pl.BlockSpec((BLK, D), lambda i: (i, 0)),
            pl.BlockSpec((1, D), lambda i: (0, 0)),
            pl.BlockSpec((D, D), lambda i: (0, 0)),
            pl.BlockSpec((1, D), lambda i: (0, 0)),
            pl.BlockSpec((D, D), lambda i: (0, 0)),
            pl.BlockSpec((1, D), lambda i: (0, 0)),
        ],
        out_specs=[
            pl.BlockSpec((BLK, D), lambda i: (i, 0)),
            pl.BlockSpec((BLK, D), lambda i: (i, 0)),
        ],
        out_shape=[
            jax.ShapeDtypeStruct((M, D), jnp.float32),
            jax.ShapeDtypeStruct((M, D), jnp.float32),
        ],
    )(h0_pre, colsum, Wh_W, Wh_b2, Wt_W, Wt_b2)


def _topk_body(eh_ref, et_ref, idx_ref):
    logits = lax.dot_general(eh_ref[...] * SCALE, et_ref[...],
                             (((1,), (1,)), ((), ())),
                             preferred_element_type=jnp.float32)
    col = lax.broadcasted_iota(jnp.int32, (BLK, M), 1)
    lane = lax.broadcasted_iota(jnp.int32, (BLK, KPAD), 1)
    work = logits
    acc = jnp.zeros((BLK, KPAD), jnp.int32)
    for k in range(K):
        m = jnp.max(work, axis=1, keepdims=True)
        idx_k = jnp.min(jnp.where(work == m, col, M), axis=1, keepdims=True)
        acc = jnp.where(lane == k, idx_k, acc)
        work = jnp.where(col == idx_k, NEG, work)
    idx_ref[...] = acc


def _run_topk(e_h, e_t):
    return pl.pallas_call(
        _topk_body,
        grid=(NBLK,),
        in_specs=[
            pl.BlockSpec((BLK, D), lambda i: (i, 0)),
            pl.BlockSpec((M, D), lambda i: (0, 0)),
        ],
        out_specs=pl.BlockSpec((BLK, KPAD), lambda i: (i, 0)),
        out_shape=jax.ShapeDtypeStruct((M, KPAD), jnp.int32),
    )(e_h, e_t)


_SC_NC = 2
_SC_NS = 16
_NROW = K * M
_PER_W = _NROW // (_SC_NC * _SC_NS)
_CHUNK = 128
_NCHUNK = _PER_W // _CHUNK


def _gather_sc(table, idx_flat):
    mesh = plsc.VectorSubcoreMesh(core_axis_name="c", subcore_axis_name="s",
                                  num_cores=_SC_NC, num_subcores=_SC_NS)

    @functools.partial(
        pl.kernel,
        out_type=jax.ShapeDtypeStruct((_NROW, D), jnp.float32),
        mesh=mesh,
        scratch_types=[
            pltpu.VMEM((_CHUNK,), jnp.int32),
            pltpu.VMEM((_CHUNK, D), jnp.float32),
            pltpu.SemaphoreType.DMA,
        ],
    )
    def gather_kernel(table_hbm, idx_hbm, out_hbm, idx_v, rows_v, sem):
        wid = lax.axis_index("s") * _SC_NC + lax.axis_index("c")
        base = wid * _PER_W
        for c in range(_NCHUNK):
            off = base + c * _CHUNK
            pltpu.sync_copy(idx_hbm.at[pl.ds(off, _CHUNK)], idx_v)
            pltpu.async_copy(table_hbm.at[idx_v], rows_v, sem).wait()
            pltpu.sync_copy(rows_v, out_hbm.at[pl.ds(off, _CHUNK)])

    return gather_kernel(table, idx_flat)


def _fuse_body(eh_ref, n0, n1, n2, n3, n4, n5,
               w1_ref, b1_ref, w2_ref, b2_ref, g_ref, bb_ref,
               out_ref, acc_ref):
    i = pl.program_id(0)
    eh = eh_ref[...]
    ns = [n0[...], n1[...], n2[...], n3[...], n4[...], n5[...]]

    w = [jnp.sum(eh * n, axis=1, keepdims=True) * SCALE for n in ns]
    mx = w[0]
    for k in range(1, K):
        mx = jnp.maximum(mx, w[k])
    ew = [jnp.exp(wk - mx) for wk in w]
    z = ew[0]
    for k in range(1, K):
        z = z + ew[k]
    p = [e / z for e in ew]

    a = [jnp.sum(n, axis=1, keepdims=True)
         * jnp.sum(jnp.tanh(pk * n + (2.0 - pk) * eh), axis=1, keepdims=True)
         for n, pk in zip(ns, p)]
    mx2 = a[0]
    for k in range(1, K):
        mx2 = jnp.maximum(mx2, a[k])
    ea = [jnp.exp(ak - mx2) for ak in a]
    z2 = ea[0]
    for k in range(1, K):
        z2 = z2 + ea[k]

    e_nh = (ea[0] / z2) * ns[0]
    for k in range(1, K):
        e_nh = e_nh + (ea[k] / z2) * ns[k]

    emb = (_leaky(_mm(eh + e_nh, w1_ref[...]) + b1_ref[...])
           + _leaky(_mm(eh * e_nh, w2_ref[...]) + b2_ref[...]))

    @pl.when(i == 0)
    def _():
        acc_ref[...] = jnp.zeros_like(acc_ref)

    acc_ref[...] += jnp.sum(emb, axis=0, keepdims=True)

    @pl.when(i == NBLK - 1)
    def _():
        h = acc_ref[...] * (1.0 / M)
        mu = jnp.mean(h, axis=1, keepdims=True)
        var = jnp.mean((h - mu) ** 2, axis=1, keepdims=True)
        out_ref[...] = ((h - mu) * lax.rsqrt(var + 1e-5) * g_ref[...]
                        + bb_ref[...])


def _run_fuse(e_h, nb, lin1_W, lin1_b2, lin2_W, lin2_b2, ln_g2, ln_b2):
    def nb_spec(k):
        return pl.BlockSpec((BLK, D), lambda i, k=k: (k * NBLK + i, 0))

    return pl.pallas_call(
        _fuse_body,
        grid=(NBLK,),
        in_specs=[
            pl.BlockSpec((BLK, D), lambda i: (i, 0)),
            nb_spec(0), nb_spec(1), nb_spec(2), nb_spec(3), nb_spec(4),
            nb_spec(5),
            pl.BlockSpec((D, D), lambda i: (0, 0)),
            pl.BlockSpec((1, D), lambda i: (0, 0)),
            pl.BlockSpec((D, D), lambda i: (0, 0)),
            pl.BlockSpec((1, D), lambda i: (0, 0)),
            pl.BlockSpec((1, D), lambda i: (0, 0)),
            pl.BlockSpec((1, D), lambda i: (0, 0)),
        ],
        out_specs=pl.BlockSpec((1, D), lambda i: (0, 0)),
        out_shape=jax.ShapeDtypeStruct((1, D), jnp.float32),
        scratch_shapes=[pltpu.VMEM((1, D), jnp.float32)],
    )(e_h, nb, nb, nb, nb, nb, nb,
      lin1_W, lin1_b2, lin2_W, lin2_b2, ln_g2, ln_b2)


def kernel(x, fc1_W, fc1_b, Wh_W, Wh_b, Wt_W, Wt_b,
           lin1_W, lin1_b, lin2_W, lin2_b, ln_g, ln_b):
    x2 = x.reshape(M, IN_DIM)
    fc1_b2 = fc1_b.reshape(1, D)
    Wh_b2 = Wh_b.reshape(1, D)
    Wt_b2 = Wt_b.reshape(1, D)
    lin1_b2 = lin1_b.reshape(1, D)
    lin2_b2 = lin2_b.reshape(1, D)
    ln_g2 = ln_g.reshape(1, D)
    ln_b2 = ln_b.reshape(1, D)

    h0_pre, colsum = _run_fc1(x2, fc1_W, fc1_b2)
    e_h, e_t = _run_ehet(h0_pre, colsum, Wh_W, Wh_b2, Wt_W, Wt_b2)
    idx8 = _run_topk(e_h, e_t)
    idx_flat = idx8[:, :K].T.reshape(_NROW)
    nb = _gather_sc(e_t, idx_flat)
    out = _run_fuse(e_h, nb, lin1_W, lin1_b2, lin2_W, lin2_b2, ln_g2, ln_b2)
    return out.reshape(D)
